# Initial kernel scaffold; baseline (speedup 1.0000x reference)
import jax
import jax.numpy as jnp
from jax.experimental import pallas as pl


def kernel(x, weights):
    raise NotImplementedError("write your pallas kernel here")


# speedup vs baseline: 2.0405x; 2.0405x over previous
import functools

import jax
import jax.numpy as jnp
from jax import lax
from jax.experimental import pallas as pl
from jax.experimental.pallas import tpu as pltpu
from jax.experimental.pallas import tpu_sc as plsc

ENC_DIM = 1024
PAD = 1
BATCH = 4
SEQ = 4096
N = BATCH * SEQ
NUM_CORES = 2
NUM_SUBCORES = 16
NUM_WORKERS = NUM_CORES * NUM_SUBCORES
CHUNK = N // NUM_WORKERS
TILES_PER_ROW = NUM_SUBCORES // (BATCH // NUM_CORES)
GROWS = 32
NGC = CHUNK // GROWS
LANES = 16


def _body(x_hbm, w_hbm, out_hbm,
          xv, idxv, cntv, allcnt, counts_sh, buf0, buf1, gsem, ssem):
    c = lax.axis_index("c")
    s = lax.axis_index("s")
    g = c * NUM_SUBCORES + s
    base = g * CHUNK

    pltpu.sync_copy(x_hbm.at[pl.ds(base, CHUNK)], xv)

    total = jnp.int32(0)
    for i in range(CHUNK // LANES):
        v = xv[pl.ds(i * LANES, LANES)]
        m = jnp.where(v != PAD, jnp.int32(1), jnp.int32(0))
        cs = jnp.cumsum(m)
        idxv[pl.ds(i * LANES, LANES)] = (cs + total) * m + PAD
        total = total + jnp.sum(m)

    cntv[...] = jnp.full((LANES,), total, jnp.int32)
    pltpu.sync_copy(cntv, counts_sh.at[s])
    plsc.subcore_barrier()
    pltpu.sync_copy(counts_sh, allcnt)
    row_start = (s // TILES_PER_ROW) * TILES_PER_ROW
    prefix = jnp.int32(0)
    for k in range(NUM_SUBCORES):
        ck = jnp.max(allcnt[k])
        take = jnp.logical_and(k >= row_start, k < s)
        prefix = prefix + jnp.where(take, ck, jnp.int32(0))

    for i in range(CHUNK // LANES):
        v = xv[pl.ds(i * LANES, LANES)]
        m = jnp.where(v != PAD, jnp.int32(1), jnp.int32(0))
        p = idxv[pl.ds(i * LANES, LANES)]
        idxv[pl.ds(i * LANES, LANES)] = p + prefix * m

    bufs = [buf0, buf1]

    def start_gather(k, buf):
        return pltpu.async_copy(
            w_hbm.at[idxv.at[pl.ds(k * GROWS, GROWS)]], buf, gsem)

    def start_scatter(k, buf):
        return pltpu.async_copy(
            buf, out_hbm.at[pl.ds(base + k * GROWS, GROWS)], ssem)

    gathers = [None] * NGC
    scatters = [None] * NGC
    gathers[0] = start_gather(0, bufs[0])
    for k in range(NGC):
        if k + 1 < NGC:
            if k >= 1:
                scatters[k - 1].wait()
            gathers[k + 1] = start_gather(k + 1, bufs[(k + 1) % 2])
        gathers[k].wait()
        scatters[k] = start_scatter(k, bufs[k % 2])
    scatters[NGC - 2].wait()
    scatters[NGC - 1].wait()


@functools.partial(jax.jit)
def kernel(x, weights):
    x_flat = x.reshape(-1).astype(jnp.int32)
    weights = weights.astype(jnp.float32)
    mesh = plsc.VectorSubcoreMesh(core_axis_name="c", subcore_axis_name="s")
    run = pl.kernel(
        _body,
        mesh=mesh,
        compiler_params=pltpu.CompilerParams(needs_layout_passes=False),
        out_type=jax.ShapeDtypeStruct((N, ENC_DIM), jnp.float32),
        scratch_types=[
            pltpu.VMEM((CHUNK,), jnp.int32),
            pltpu.VMEM((CHUNK,), jnp.int32),
            pltpu.VMEM((LANES,), jnp.int32),
            pltpu.VMEM((NUM_SUBCORES, LANES), jnp.int32),
            pltpu.VMEM_SHARED((NUM_SUBCORES, LANES), jnp.int32),
            pltpu.VMEM((GROWS, ENC_DIM), jnp.float32),
            pltpu.VMEM((GROWS, ENC_DIM), jnp.float32),
            pltpu.SemaphoreType.DMA,
            pltpu.SemaphoreType.DMA,
        ],
    )
    out = run(x_flat, weights)
    return lax.stop_gradient(out.reshape(BATCH, SEQ, ENC_DIM))


# speedup vs baseline: 2.0413x; 1.0004x over previous
import functools

import jax
import jax.numpy as jnp
from jax import lax
from jax.experimental import pallas as pl
from jax.experimental.pallas import tpu as pltpu
from jax.experimental.pallas import tpu_sc as plsc

ENC_DIM = 1024
PAD = 1
BATCH = 4
SEQ = 4096
N = BATCH * SEQ
NUM_CORES = 2
NUM_SUBCORES = 16
NUM_WORKERS = NUM_CORES * NUM_SUBCORES
CHUNK = N // NUM_WORKERS
TILES_PER_ROW = NUM_SUBCORES // (BATCH // NUM_CORES)
GROWS = 32
NGC = CHUNK // GROWS
NBUF = 3
LANES = 16


def _body(x_hbm, w_hbm, out_hbm,
          xv, idxv, cntv, allcnt, counts_sh, buf0, buf1, buf2, gsem, ssem):
    c = lax.axis_index("c")
    s = lax.axis_index("s")
    g = c * NUM_SUBCORES + s
    base = g * CHUNK

    pltpu.sync_copy(x_hbm.at[pl.ds(base, CHUNK)], xv)

    total = jnp.int32(0)
    for i in range(CHUNK // LANES):
        v = xv[pl.ds(i * LANES, LANES)]
        m = jnp.where(v != PAD, jnp.int32(1), jnp.int32(0))
        cs = jnp.cumsum(m)
        idxv[pl.ds(i * LANES, LANES)] = (cs + total) * m + PAD
        total = total + jnp.sum(m)

    cntv[...] = jnp.full((LANES,), total, jnp.int32)
    pltpu.sync_copy(cntv, counts_sh.at[s])
    plsc.subcore_barrier()
    pltpu.sync_copy(counts_sh, allcnt)
    row_start = (s // TILES_PER_ROW) * TILES_PER_ROW
    prefix = jnp.int32(0)
    for k in range(NUM_SUBCORES):
        ck = jnp.max(allcnt[k])
        take = jnp.logical_and(k >= row_start, k < s)
        prefix = prefix + jnp.where(take, ck, jnp.int32(0))

    for i in range(CHUNK // LANES):
        v = xv[pl.ds(i * LANES, LANES)]
        m = jnp.where(v != PAD, jnp.int32(1), jnp.int32(0))
        p = idxv[pl.ds(i * LANES, LANES)]
        idxv[pl.ds(i * LANES, LANES)] = p + prefix * m

    bufs = [buf0, buf1, buf2]

    def start_gather(k, buf):
        return pltpu.async_copy(
            w_hbm.at[idxv.at[pl.ds(k * GROWS, GROWS)]], buf, gsem)

    def start_scatter(k, buf):
        return pltpu.async_copy(
            buf, out_hbm.at[pl.ds(base + k * GROWS, GROWS)], ssem)

    gathers = [None] * NGC
    scatters = [None] * NGC
    gathers[0] = start_gather(0, bufs[0])
    gathers[1] = start_gather(1, bufs[1])
    for k in range(NGC):
        if k + 2 < NGC:
            if k >= 1:
                scatters[k - 1].wait()
            gathers[k + 2] = start_gather(k + 2, bufs[(k + 2) % NBUF])
        gathers[k].wait()
        scatters[k] = start_scatter(k, bufs[k % NBUF])
    scatters[NGC - 3].wait()
    scatters[NGC - 2].wait()
    scatters[NGC - 1].wait()


@functools.partial(jax.jit)
def kernel(x, weights):
    x_flat = x.reshape(-1).astype(jnp.int32)
    weights = weights.astype(jnp.float32)
    mesh = plsc.VectorSubcoreMesh(core_axis_name="c", subcore_axis_name="s")
    run = pl.kernel(
        _body,
        mesh=mesh,
        compiler_params=pltpu.CompilerParams(needs_layout_passes=False),
        out_type=jax.ShapeDtypeStruct((N, ENC_DIM), jnp.float32),
        scratch_types=[
            pltpu.VMEM((CHUNK,), jnp.int32),
            pltpu.VMEM((CHUNK,), jnp.int32),
            pltpu.VMEM((LANES,), jnp.int32),
            pltpu.VMEM((NUM_SUBCORES, LANES), jnp.int32),
            pltpu.VMEM_SHARED((NUM_SUBCORES, LANES), jnp.int32),
            pltpu.VMEM((GROWS, ENC_DIM), jnp.float32),
            pltpu.VMEM((GROWS, ENC_DIM), jnp.float32),
            pltpu.VMEM((GROWS, ENC_DIM), jnp.float32),
            pltpu.SemaphoreType.DMA,
            pltpu.SemaphoreType.DMA,
        ],
    )
    out = run(x_flat, weights)
    return lax.stop_gradient(out.reshape(BATCH, SEQ, ENC_DIM))
